# baseline (speedup 1.0000x reference)
import functools

import jax
import jax.numpy as jnp
from jax import lax
from jax.experimental import pallas as pl
from jax.experimental.pallas import tpu as pltpu
from jax.experimental.pallas import tpu_sc as plsc

EMB_DIM = 32
BATCH = 16384
NC = 2
NS = 16
NW = NC * NS
B_PER_W = BATCH // NW
L = 16
GCHUNK = 128
NCHUNK = B_PER_W // GCHUNK


def _rsqrt16(x):
    i = lax.bitcast_convert_type(x, jnp.int32)
    y = lax.bitcast_convert_type(jnp.int32(0x5F3759DF) - (i >> 1), jnp.float32)
    for _ in range(3):
        y = y * (jnp.float32(1.5) - jnp.float32(0.5) * x * y * y)
    return y


def _make_kernel():
    mesh = plsc.VectorSubcoreMesh(core_axis_name="c", subcore_axis_name="s")

    @functools.partial(
        pl.kernel,
        mesh=mesh,
        compiler_params=pltpu.CompilerParams(
            needs_layout_passes=False, use_tc_tiling_on_sc=False),
        out_type=jax.ShapeDtypeStruct((BATCH,), jnp.float32),
        scratch_types=[
            pltpu.VMEM((B_PER_W,), jnp.int32),
            pltpu.VMEM((B_PER_W,), jnp.int32),
            pltpu.VMEM((EMB_DIM, B_PER_W), jnp.float32),
            pltpu.VMEM((EMB_DIM, B_PER_W), jnp.float32),
            pltpu.VMEM((B_PER_W,), jnp.float32),
            pltpu.SemaphoreType.DMA,
            pltpu.SemaphoreType.DMA,
            pltpu.SemaphoreType.DMA,
        ],
    )
    def cosine_kernel(uids_hbm, jids_hbm, utab_hbm, jtab_hbm, out_hbm,
                      uidx_v, jidx_v, urows_v, jrows_v, outv,
                      sem_i, sem_u, sem_j):
        wid = lax.axis_index("s") * NC + lax.axis_index("c")
        base = wid * B_PER_W

        ci_u = pltpu.async_copy(uids_hbm.at[pl.ds(base, B_PER_W)], uidx_v, sem_i)
        ci_j = pltpu.async_copy(jids_hbm.at[pl.ds(base, B_PER_W)], jidx_v, sem_i)
        ci_u.wait()
        ci_j.wait()

        copies = []
        for f in range(EMB_DIM):
            for c in range(NCHUNK):
                sl = pl.ds(c * GCHUNK, GCHUNK)
                copies.append(pltpu.async_copy(
                    utab_hbm.at[f].at[uidx_v.at[sl]],
                    urows_v.at[f].at[sl], sem_u))
                copies.append(pltpu.async_copy(
                    jtab_hbm.at[f].at[jidx_v.at[sl]],
                    jrows_v.at[f].at[sl], sem_j))
        for cp in copies:
            cp.wait()

        zeros = jnp.zeros((L,), jnp.float32)
        eps = jnp.float32(1e-12)

        def group_body(g, carry):
            d = zeros
            uu = zeros
            jj = zeros
            for f in range(EMB_DIM):
                u = urows_v[f, pl.ds(g * L, L)]
                v = jrows_v[f, pl.ds(g * L, L)]
                d = d + u * v
                uu = uu + u * u
                jj = jj + v * v
            uu = jnp.maximum(uu, eps)
            jj = jnp.maximum(jj, eps)
            outv[pl.ds(g * L, L)] = d * _rsqrt16(uu) * _rsqrt16(jj)
            return carry

        lax.fori_loop(0, B_PER_W // L, group_body, 0)
        pltpu.sync_copy(outv, out_hbm.at[pl.ds(base, B_PER_W)])

    return cosine_kernel


_kernel_call = _make_kernel()


def kernel(user_ids, joke_ids, user_table, joke_table):
    out = _kernel_call(user_ids, joke_ids, user_table.T, joke_table.T)
    return out.reshape(BATCH, 1)


# speedup vs baseline: 13.6835x; 13.6835x over previous
import functools

import jax
import jax.numpy as jnp
from jax import lax
from jax.experimental import pallas as pl
from jax.experimental.pallas import tpu as pltpu
from jax.experimental.pallas import tpu_sc as plsc

EMB_DIM = 32
BATCH = 16384
NC = 2
NS = 16
NW = NC * NS
B_PER_W = BATCH // NW
L = 16
NWAVE = B_PER_W // L


def _rsqrt16(x):
    i = lax.bitcast_convert_type(x, jnp.int32)
    y = lax.bitcast_convert_type(jnp.int32(0x5F3759DF) - (i >> 1), jnp.float32)
    for _ in range(3):
        y = y * (jnp.float32(1.5) - jnp.float32(0.5) * x * y * y)
    return y


def _make_kernel():
    mesh = plsc.VectorSubcoreMesh(core_axis_name="c", subcore_axis_name="s")

    @functools.partial(
        pl.kernel,
        mesh=mesh,
        compiler_params=pltpu.CompilerParams(needs_layout_passes=False),
        out_type=jax.ShapeDtypeStruct((BATCH,), jnp.float32),
        scratch_types=[
            pltpu.VMEM((B_PER_W,), jnp.int32),
            pltpu.VMEM((B_PER_W,), jnp.int32),
            pltpu.VMEM((L * EMB_DIM, 128), jnp.float32),
            pltpu.VMEM((L * 8, EMB_DIM), jnp.float32),
            pltpu.VMEM((B_PER_W,), jnp.float32),
            pltpu.SemaphoreType.DMA,
            pltpu.SemaphoreType.DMA,
            pltpu.SemaphoreType.DMA,
        ],
    )
    def cosine_kernel(uids_hbm, jids_hbm, utabt_hbm, jtab_hbm, out_hbm,
                      uidx_v, jidx_v, uwin_v, jblk_v, outv,
                      sem_i, sem_u, sem_j):
        wid = lax.axis_index("s") * NC + lax.axis_index("c")
        base = wid * B_PER_W

        ci_u = pltpu.async_copy(uids_hbm.at[pl.ds(base, B_PER_W)], uidx_v, sem_i)
        ci_j = pltpu.async_copy(jids_hbm.at[pl.ds(base, B_PER_W)], jidx_v, sem_i)
        ci_u.wait()
        ci_j.wait()

        iota16 = lax.iota(jnp.int32, 16)
        zeros = jnp.zeros((L,), jnp.float32)
        eps = jnp.float32(1e-12)

        def wave_body(w, carry):
            wbase = w * L
            uvec = uidx_v[pl.ds(wbase, L)]
            jvec = jidx_v[pl.ds(wbase, L)]

            for k in range(L):
                u = uvec[k]
                j = jvec[k]
                uw = pl.multiple_of((u >> 7) << 7, 128)
                udst = pl.multiple_of(k * EMB_DIM, 8)
                pltpu.async_copy(
                    utabt_hbm.at[:, pl.ds(uw, 128)],
                    uwin_v.at[pl.ds(udst, EMB_DIM), :], sem_u)
                jb = pl.multiple_of((j >> 3) << 3, 8)
                jdst = pl.multiple_of(k * 8, 8)
                pltpu.async_copy(
                    jtab_hbm.at[pl.ds(jb, 8), :],
                    jblk_v.at[pl.ds(jdst, 8), :], sem_j)

            pltpu.make_async_copy(
                utabt_hbm.at[:, pl.ds(0, L * 128)].at[pl.ds(0, L * EMB_DIM), :],
                uwin_v, sem_u).wait()
            pltpu.make_async_copy(
                jtab_hbm.at[pl.ds(0, L * 8), :], jblk_v, sem_j).wait()

            ucol = uvec & 127
            jrow = iota16 * 8 + (jvec & 7)
            d = zeros
            uu = zeros
            jj = zeros
            for f in range(EMB_DIM):
                urow = iota16 * EMB_DIM + f
                uf = plsc.load_gather(uwin_v, [urow, ucol])
                jf = plsc.load_gather(jblk_v, [jrow, jnp.full((L,), f, jnp.int32)])
                d = d + uf * jf
                uu = uu + uf * uf
                jj = jj + jf * jf
            uu = jnp.maximum(uu, eps)
            jj = jnp.maximum(jj, eps)
            outv[pl.ds(wbase, L)] = d * _rsqrt16(uu) * _rsqrt16(jj)
            return carry

        lax.fori_loop(0, NWAVE, wave_body, 0)
        pltpu.sync_copy(outv, out_hbm.at[pl.ds(base, B_PER_W)])

    return cosine_kernel


_kernel_call = _make_kernel()


def kernel(user_ids, joke_ids, user_table, joke_table):
    out = _kernel_call(user_ids, joke_ids, user_table.T, joke_table)
    return out.reshape(BATCH, 1)


# speedup vs baseline: 14.5256x; 1.0615x over previous
import functools

import jax
import jax.numpy as jnp
from jax import lax
from jax.experimental import pallas as pl
from jax.experimental.pallas import tpu as pltpu
from jax.experimental.pallas import tpu_sc as plsc

EMB_DIM = 32
BATCH = 16384
NC = 2
NS = 16
NW = NC * NS
B_PER_W = BATCH // NW
L = 16
H = 8
NHALF = B_PER_W // H


def _rsqrt16(x):
    i = lax.bitcast_convert_type(x, jnp.int32)
    y = lax.bitcast_convert_type(jnp.int32(0x5F3759DF) - (i >> 1), jnp.float32)
    for _ in range(3):
        y = y * (jnp.float32(1.5) - jnp.float32(0.5) * x * y * y)
    return y


def _make_kernel():
    mesh = plsc.VectorSubcoreMesh(core_axis_name="c", subcore_axis_name="s")

    @functools.partial(
        pl.kernel,
        mesh=mesh,
        compiler_params=pltpu.CompilerParams(needs_layout_passes=False),
        out_type=jax.ShapeDtypeStruct((BATCH,), jnp.float32),
        scratch_types=[
            pltpu.VMEM((B_PER_W + L,), jnp.int32),
            pltpu.VMEM((B_PER_W + L,), jnp.int32),
            pltpu.VMEM((H * EMB_DIM, 128), jnp.float32),
            pltpu.VMEM((H * EMB_DIM, 128), jnp.float32),
            pltpu.VMEM((H * 8, EMB_DIM), jnp.float32),
            pltpu.VMEM((H * 8, EMB_DIM), jnp.float32),
            pltpu.VMEM((B_PER_W,), jnp.float32),
            pltpu.SemaphoreType.DMA,
            pltpu.SemaphoreType.DMA,
            pltpu.SemaphoreType.DMA,
            pltpu.SemaphoreType.DMA,
            pltpu.SemaphoreType.DMA,
        ],
    )
    def cosine_kernel(uids_hbm, jids_hbm, utabt_hbm, jtab_hbm, out_hbm,
                      uidx_v, jidx_v, uwin_a, uwin_b, jblk_a, jblk_b, outv,
                      sem_i, sem_ua, sem_ub, sem_ja, sem_jb):
        wid = lax.axis_index("s") * NC + lax.axis_index("c")
        base = wid * B_PER_W

        ci_u = pltpu.async_copy(uids_hbm.at[pl.ds(base, B_PER_W)],
                                uidx_v.at[pl.ds(0, B_PER_W)], sem_i)
        ci_j = pltpu.async_copy(jids_hbm.at[pl.ds(base, B_PER_W)],
                                jidx_v.at[pl.ds(0, B_PER_W)], sem_i)
        ci_u.wait()
        ci_j.wait()

        iota16 = lax.iota(jnp.int32, 16)
        lane8 = iota16 & 7
        zeros = jnp.zeros((L,), jnp.float32)
        eps = jnp.float32(1e-12)

        def enqueue_half(h, uwin, jblk, sem_u, sem_j):
            uvec = uidx_v[pl.ds(h * H, L)]
            jvec = jidx_v[pl.ds(h * H, L)]
            for k in range(H):
                u = uvec[k]
                j = jvec[k]
                uw = pl.multiple_of((u >> 7) << 7, 128)
                udst = pl.multiple_of(k * EMB_DIM, 8)
                pltpu.async_copy(
                    utabt_hbm.at[:, pl.ds(uw, 128)],
                    uwin.at[pl.ds(udst, EMB_DIM), :], sem_u)
                jb = pl.multiple_of((j >> 3) << 3, 8)
                jdst = pl.multiple_of(k * 8, 8)
                pltpu.async_copy(
                    jtab_hbm.at[pl.ds(jb, 8), :],
                    jblk.at[pl.ds(jdst, 8), :], sem_j)

        def drain_half(uwin, jblk, sem_u, sem_j):
            pltpu.make_async_copy(
                utabt_hbm.at[:, pl.ds(0, H * 128)].at[pl.ds(0, H * EMB_DIM), :],
                uwin, sem_u).wait()
            pltpu.make_async_copy(
                jtab_hbm.at[pl.ds(0, H * 8), :], jblk, sem_j).wait()

        def extract_half(h, uwin, jblk, rev):
            elem = (7 - lane8) if rev else lane8
            uvec = uidx_v[pl.ds(h * H, L)]
            jvec = jidx_v[pl.ds(h * H, L)]
            du = plsc.load_gather(uidx_v, [h * H + elem])
            dj = plsc.load_gather(jidx_v, [h * H + elem])
            del uvec, jvec
            ucol = du & 127
            jrow = elem * 8 + (dj & 7)
            d = zeros
            uu = zeros
            jj = zeros
            for f in range(EMB_DIM):
                urow = elem * EMB_DIM + f
                uf = plsc.load_gather(uwin, [urow, ucol])
                jf = plsc.load_gather(jblk, [jrow, jnp.full((L,), f, jnp.int32)])
                d = d + uf * jf
                uu = uu + uf * uf
                jj = jj + jf * jf
            uu = jnp.maximum(uu, eps)
            jj = jnp.maximum(jj, eps)
            return d * _rsqrt16(uu) * _rsqrt16(jj)

        enqueue_half(0, uwin_a, jblk_a, sem_ua, sem_ja)

        def pair_body(p, carry):
            h = p * 2
            enqueue_half(h + 1, uwin_b, jblk_b, sem_ub, sem_jb)
            drain_half(uwin_a, jblk_a, sem_ua, sem_ja)
            r_low = extract_half(h, uwin_a, jblk_a, rev=False)

            @pl.when(p < NHALF // 2 - 1)
            def _():
                enqueue_half(h + 2, uwin_a, jblk_a, sem_ua, sem_ja)

            drain_half(uwin_b, jblk_b, sem_ub, sem_jb)
            r_high = extract_half(h + 1, uwin_b, jblk_b, rev=True)
            merged = jnp.where(iota16 < 8, r_low, lax.rev(r_high, (0,)))
            outv[pl.ds(p * L, L)] = merged
            return carry

        lax.fori_loop(0, NHALF // 2, pair_body, 0)
        pltpu.sync_copy(outv, out_hbm.at[pl.ds(base, B_PER_W)])

    return cosine_kernel


_kernel_call = _make_kernel()


def kernel(user_ids, joke_ids, user_table, joke_table):
    out = _kernel_call(user_ids, joke_ids, user_table.T, joke_table)
    return out.reshape(BATCH, 1)
